# Initial kernel scaffold; baseline (speedup 1.0000x reference)
import jax
import jax.numpy as jnp
from jax.experimental import pallas as pl


def kernel(input_i_1SBH, gate_W, w1, w2, w3):
    raise NotImplementedError("write your pallas kernel here")


# speedup vs baseline: 1.1871x; 1.1871x over previous
import jax
import jax.numpy as jnp
from jax import lax
from jax.experimental import pallas as pl
from jax.experimental.pallas import tpu as pltpu

D_MODEL = 2048
D_FF = 4096
E = 8
B = 32
LANES = 128
FF_BLK = 512
NF = D_FF // FF_BLK


def _moe_kernel(x_ref, gw_ref, w1_ref, w3_ref, w2_ref, out_ref, coeff_ref):
    e = pl.program_id(0)
    f = pl.program_id(1)
    iota = lax.broadcasted_iota(jnp.int32, (B, LANES), 1)

    @pl.when((e == 0) & (f == 0))
    def _gate_and_init():
        x = x_ref[...]
        logits = jnp.dot(x, gw_ref[...], preferred_element_type=jnp.float32)
        neg = jnp.float32(-jnp.inf)
        logits = jnp.where(iota < E, logits, neg)
        m1 = jnp.max(logits, axis=1, keepdims=True)
        i1 = jnp.min(jnp.where(logits == m1, iota, LANES), axis=1, keepdims=True)
        l2 = jnp.where(iota == i1, neg, logits)
        m2 = jnp.max(l2, axis=1, keepdims=True)
        i2 = jnp.min(jnp.where(l2 == m2, iota, LANES), axis=1, keepdims=True)
        z = jnp.exp(m2 - m1)
        p1 = 1.0 / (1.0 + z)
        p2 = 1.0 - p1
        coeff_ref[...] = (jnp.where(iota == i1, p1, 0.0)
                          + jnp.where(iota == i2, p2, 0.0))
        out_ref[...] = jnp.zeros_like(out_ref)

    x = x_ref[...]
    h = jax.nn.silu(jnp.dot(x, w1_ref[0], preferred_element_type=jnp.float32))
    h = h * jnp.dot(x, w3_ref[0], preferred_element_type=jnp.float32)
    c = jnp.sum(jnp.where(iota == e, coeff_ref[...], 0.0), axis=1, keepdims=True)
    out_ref[...] += jnp.dot(h * c, w2_ref[0], preferred_element_type=jnp.float32)


def kernel(input_i_1SBH, gate_W, w1, w2, w3):
    x = input_i_1SBH.reshape(B, D_MODEL)
    gw = jnp.pad(gate_W, ((0, 0), (0, LANES - E)))
    out = pl.pallas_call(
        _moe_kernel,
        grid=(E, NF),
        in_specs=[
            pl.BlockSpec((B, D_MODEL), lambda e, f: (0, 0)),
            pl.BlockSpec((D_MODEL, LANES), lambda e, f: (0, 0)),
            pl.BlockSpec((1, D_MODEL, FF_BLK), lambda e, f: (e, 0, f)),
            pl.BlockSpec((1, D_MODEL, FF_BLK), lambda e, f: (e, 0, f)),
            pl.BlockSpec((1, FF_BLK, D_MODEL), lambda e, f: (e, f, 0)),
        ],
        out_specs=pl.BlockSpec((B, D_MODEL), lambda e, f: (0, 0)),
        out_shape=jax.ShapeDtypeStruct((B, D_MODEL), jnp.float32),
        scratch_shapes=[pltpu.VMEM((B, LANES), jnp.float32)],
        compiler_params=pltpu.CompilerParams(
            dimension_semantics=("arbitrary", "arbitrary"),
        ),
    )(x, gw, w1, w3, w2)
    return out.reshape(input_i_1SBH.shape)
